# Initial kernel scaffold; baseline (speedup 1.0000x reference)
import jax
import jax.numpy as jnp
from jax.experimental import pallas as pl


def kernel(x, pe_table):
    raise NotImplementedError("write your pallas kernel here")


# speedup vs baseline: 1.6798x; 1.6798x over previous
import jax
import jax.numpy as jnp
from jax.experimental import pallas as pl

_S_BLK = 512


def _add_pe_kernel(x_ref, pe_ref, o_ref):
    o_ref[...] = x_ref[...] + pe_ref[...][None, :, :]


def kernel(x, pe_table):
    B, S, D = x.shape
    n_s = S // _S_BLK
    return pl.pallas_call(
        _add_pe_kernel,
        grid=(n_s, B),
        in_specs=[
            pl.BlockSpec((1, _S_BLK, D), lambda i, b: (b, i, 0)),
            pl.BlockSpec((_S_BLK, D), lambda i, b: (i, 0)),
        ],
        out_specs=pl.BlockSpec((1, _S_BLK, D), lambda i, b: (b, i, 0)),
        out_shape=jax.ShapeDtypeStruct((B, S, D), x.dtype),
    )(x, pe_table)


# speedup vs baseline: 1.8534x; 1.1033x over previous
import jax
import jax.numpy as jnp
from jax.experimental import pallas as pl

_S_BLK = 1024


def _add_pe_kernel(x_ref, pe_ref, o_ref):
    o_ref[...] = x_ref[...] + pe_ref[...][None, :, :]


def kernel(x, pe_table):
    B, S, D = x.shape
    n_s = S // _S_BLK
    return pl.pallas_call(
        _add_pe_kernel,
        grid=(n_s, B),
        in_specs=[
            pl.BlockSpec((1, _S_BLK, D), lambda i, b: (b, i, 0)),
            pl.BlockSpec((_S_BLK, D), lambda i, b: (i, 0)),
        ],
        out_specs=pl.BlockSpec((1, _S_BLK, D), lambda i, b: (b, i, 0)),
        out_shape=jax.ShapeDtypeStruct((B, S, D), x.dtype),
    )(x, pe_table)


# speedup vs baseline: 1.9630x; 1.0591x over previous
import jax
import jax.numpy as jnp
from jax.experimental import pallas as pl

_S_BLK = 2048


def _add_pe_kernel(x_ref, pe_ref, o_ref):
    o_ref[...] = x_ref[...] + pe_ref[...][None, :, :]


def kernel(x, pe_table):
    B, S, D = x.shape
    n_s = S // _S_BLK
    return pl.pallas_call(
        _add_pe_kernel,
        grid=(n_s, B),
        in_specs=[
            pl.BlockSpec((1, _S_BLK, D), lambda i, b: (b, i, 0)),
            pl.BlockSpec((_S_BLK, D), lambda i, b: (i, 0)),
        ],
        out_specs=pl.BlockSpec((1, _S_BLK, D), lambda i, b: (b, i, 0)),
        out_shape=jax.ShapeDtypeStruct((B, S, D), x.dtype),
    )(x, pe_table)
